# Initial kernel scaffold; baseline (speedup 1.0000x reference)
import functools

import jax
import jax.numpy as jnp
from jax import lax
from jax.experimental import pallas as pl
from jax.experimental.pallas import tpu as pltpu
from jax.experimental.pallas import tpu_sc as plsc

CHAR_DIM = 32
WORD_DIM = 32
OUT_DIM = CHAR_DIM + WORD_DIM
IPR = 128
CHUNK_ROWS = 4
CHUNK = CHUNK_ROWS * IPR


@functools.lru_cache(maxsize=None)
def _make_sc_kernel(n_tokens: int):
    info = plsc.get_sparse_core_info()
    nw = info.num_cores * info.num_subcores
    assert n_tokens % (nw * CHUNK) == 0
    rows_per_w = (n_tokens // IPR) // nw
    n_iter = rows_per_w // CHUNK_ROWS
    nc = info.num_cores

    mesh = plsc.VectorSubcoreMesh(core_axis_name="c", subcore_axis_name="s")

    @functools.partial(
        pl.kernel,
        mesh=mesh,
        out_type=jax.ShapeDtypeStruct((n_tokens, OUT_DIM), jnp.float32),
        scratch_types=[
            pltpu.VMEM((CHUNK_ROWS, IPR), jnp.int32),
            pltpu.VMEM((CHUNK_ROWS, IPR), jnp.int32),
            pltpu.VMEM((CHUNK, CHAR_DIM), jnp.float32),
            pltpu.VMEM((CHUNK, WORD_DIM), jnp.float32),
            pltpu.SemaphoreType.DMA,
        ],
    )
    def k(idx_char_hbm, idx_word_hbm, char_hbm, word_hbm, out_hbm,
          ic_v, iw_v, cb_v, wb_v, sem):
        wid = lax.axis_index("s") * nc + lax.axis_index("c")
        row_base = wid * rows_per_w
        tok_base = row_base * IPR

        def body(i, carry):
            row0 = row_base + i * CHUNK_ROWS
            tok0 = tok_base + i * CHUNK
            pltpu.sync_copy(idx_char_hbm.at[pl.ds(row0, CHUNK_ROWS)], ic_v)
            pltpu.sync_copy(idx_word_hbm.at[pl.ds(row0, CHUNK_ROWS)], iw_v)
            copies = []
            for g in range(CHUNK_ROWS):
                copies.append(pltpu.async_copy(
                    char_hbm.at[ic_v.at[g]],
                    cb_v.at[pl.ds(g * IPR, IPR)], sem))
                copies.append(pltpu.async_copy(
                    word_hbm.at[iw_v.at[g]],
                    wb_v.at[pl.ds(g * IPR, IPR)], sem))
            for c in copies:
                c.wait()
            pltpu.sync_copy(
                cb_v, out_hbm.at[pl.ds(tok0, CHUNK), pl.ds(0, CHAR_DIM)])
            pltpu.sync_copy(
                wb_v, out_hbm.at[pl.ds(tok0, CHUNK), pl.ds(CHAR_DIM, WORD_DIM)])
            return carry

        lax.fori_loop(0, n_iter, body, 0)

    return k


def kernel(voc, char_table, word_table):
    b, _, l = voc.shape
    n_tokens = b * l
    idx_char = voc[:, 0, :].reshape(n_tokens // IPR, IPR).astype(jnp.int32)
    idx_word = voc[:, 1, :].reshape(n_tokens // IPR, IPR).astype(jnp.int32)
    out = _make_sc_kernel(n_tokens)(idx_char, idx_word, char_table, word_table)
    return out.reshape(b, l, OUT_DIM)


# speedup vs baseline: 7.0185x; 7.0185x over previous
import functools

import jax
import jax.numpy as jnp
from jax import lax
from jax.experimental import pallas as pl
from jax.experimental.pallas import tpu as pltpu
from jax.experimental.pallas import tpu_sc as plsc

CHAR_DIM = 32
WORD_DIM = 32
OUT_DIM = CHAR_DIM + WORD_DIM
IPR = 128
CHUNK_ROWS = 4
CHUNK = CHUNK_ROWS * IPR


@functools.lru_cache(maxsize=None)
def _make_sc_kernel(n_tokens: int):
    info = plsc.get_sparse_core_info()
    nw = info.num_cores * info.num_subcores
    assert n_tokens % (nw * CHUNK) == 0
    rows_per_w = (n_tokens // IPR) // nw
    n_iter = rows_per_w // CHUNK_ROWS
    nc = info.num_cores

    mesh = plsc.VectorSubcoreMesh(core_axis_name="c", subcore_axis_name="s")

    @functools.partial(
        pl.kernel,
        mesh=mesh,
        out_type=jax.ShapeDtypeStruct((n_tokens, OUT_DIM), jnp.float32),
        compiler_params=pltpu.CompilerParams(use_tc_tiling_on_sc=False),
        scratch_types=[
            pltpu.VMEM((CHUNK_ROWS, IPR), jnp.int32),
            pltpu.VMEM((CHUNK_ROWS, IPR), jnp.int32),
            pltpu.VMEM((CHUNK, CHAR_DIM), jnp.float32),
            pltpu.VMEM((CHUNK, WORD_DIM), jnp.float32),
            pltpu.SemaphoreType.DMA,
        ],
    )
    def k(idx_char_hbm, idx_word_hbm, char_hbm, word_hbm, out_hbm,
          ic_v, iw_v, cb_v, wb_v, sem):
        wid = lax.axis_index("s") * nc + lax.axis_index("c")
        row_base = wid * rows_per_w
        tok_base = row_base * IPR

        def body(i, carry):
            row0 = row_base + i * CHUNK_ROWS
            tok0 = tok_base + i * CHUNK
            pltpu.sync_copy(idx_char_hbm.at[pl.ds(row0, CHUNK_ROWS)], ic_v)
            pltpu.sync_copy(idx_word_hbm.at[pl.ds(row0, CHUNK_ROWS)], iw_v)
            copies = []
            for g in range(CHUNK_ROWS):
                copies.append(pltpu.async_copy(
                    char_hbm.at[ic_v.at[g]],
                    cb_v.at[pl.ds(g * IPR, IPR)], sem))
                copies.append(pltpu.async_copy(
                    word_hbm.at[iw_v.at[g]],
                    wb_v.at[pl.ds(g * IPR, IPR)], sem))
            for c in copies:
                c.wait()
            pltpu.sync_copy(
                cb_v, out_hbm.at[pl.ds(tok0, CHUNK), pl.ds(0, CHAR_DIM)])
            pltpu.sync_copy(
                wb_v, out_hbm.at[pl.ds(tok0, CHUNK), pl.ds(CHAR_DIM, WORD_DIM)])
            return carry

        lax.fori_loop(0, n_iter, body, 0)

    return k


def kernel(voc, char_table, word_table):
    b, _, l = voc.shape
    n_tokens = b * l
    idx_char = voc[:, 0, :].reshape(n_tokens // IPR, IPR).astype(jnp.int32)
    idx_word = voc[:, 1, :].reshape(n_tokens // IPR, IPR).astype(jnp.int32)
    out = _make_sc_kernel(n_tokens)(idx_char, idx_word, char_table, word_table)
    return out.reshape(b, l, OUT_DIM)


# speedup vs baseline: 7.3645x; 1.0493x over previous
import functools

import jax
import jax.numpy as jnp
from jax import lax
from jax.experimental import pallas as pl
from jax.experimental.pallas import tpu as pltpu
from jax.experimental.pallas import tpu_sc as plsc

CHAR_DIM = 32
WORD_DIM = 32
OUT_DIM = CHAR_DIM + WORD_DIM
IPR = 128
CHUNK_ROWS = 4
CHUNK = CHUNK_ROWS * IPR


@functools.lru_cache(maxsize=None)
def _make_sc_kernel(n_tokens: int):
    info = plsc.get_sparse_core_info()
    nw = info.num_cores * info.num_subcores
    assert n_tokens % (nw * CHUNK) == 0
    rows_per_w = (n_tokens // IPR) // nw
    n_iter = rows_per_w // CHUNK_ROWS
    nc = info.num_cores

    mesh = plsc.VectorSubcoreMesh(core_axis_name="c", subcore_axis_name="s")

    @functools.partial(
        pl.kernel,
        mesh=mesh,
        out_type=jax.ShapeDtypeStruct((n_tokens, OUT_DIM), jnp.float32),
        compiler_params=pltpu.CompilerParams(use_tc_tiling_on_sc=False),
        scratch_types=[
            pltpu.VMEM((2, CHUNK_ROWS, IPR), jnp.int32),
            pltpu.VMEM((2, CHUNK_ROWS, IPR), jnp.int32),
            pltpu.VMEM((2, CHUNK, CHAR_DIM), jnp.float32),
            pltpu.VMEM((2, CHUNK, WORD_DIM), jnp.float32),
            pltpu.SemaphoreType.DMA,
            pltpu.SemaphoreType.DMA,
        ],
    )
    def k(idx_char_hbm, idx_word_hbm, char_hbm, word_hbm, out_hbm,
          ic_v, iw_v, cb_v, wb_v, sem_g, sem_w):
        wid = lax.axis_index("s") * nc + lax.axis_index("c")
        row_base = wid * rows_per_w
        tok_base = row_base * IPR

        def gather_copies(slot):
            copies = []
            for g in range(CHUNK_ROWS):
                rows = pl.ds(g * IPR, IPR)
                copies.append(pltpu.make_async_copy(
                    char_hbm.at[ic_v.at[slot, g]],
                    cb_v.at[slot, rows], sem_g))
                copies.append(pltpu.make_async_copy(
                    word_hbm.at[iw_v.at[slot, g]],
                    wb_v.at[slot, rows], sem_g))
            return copies

        def issue_gathers(chunk_i, slot):
            row0 = row_base + chunk_i * CHUNK_ROWS
            pltpu.sync_copy(idx_char_hbm.at[pl.ds(row0, CHUNK_ROWS)],
                            ic_v.at[slot])
            pltpu.sync_copy(idx_word_hbm.at[pl.ds(row0, CHUNK_ROWS)],
                            iw_v.at[slot])
            for c in gather_copies(slot):
                c.start()

        def write_copies(chunk_i, slot):
            tok0 = tok_base + chunk_i * CHUNK
            return [
                pltpu.make_async_copy(
                    cb_v.at[slot],
                    out_hbm.at[pl.ds(tok0, CHUNK), pl.ds(0, CHAR_DIM)],
                    sem_w),
                pltpu.make_async_copy(
                    wb_v.at[slot],
                    out_hbm.at[pl.ds(tok0, CHUNK), pl.ds(CHAR_DIM, WORD_DIM)],
                    sem_w),
            ]

        issue_gathers(0, 0)

        def body(i, carry):
            slot = lax.rem(i, 2)
            nslot = 1 - slot
            for c in gather_copies(slot):
                c.wait()
            for c in write_copies(i, slot):
                c.start()

            @pl.when(i > 0)
            def _drain_prev_write():
                for c in write_copies(i - 1, nslot):
                    c.wait()

            @pl.when(i + 1 < n_iter)
            def _prefetch_next():
                issue_gathers(i + 1, nslot)

            return carry

        lax.fori_loop(0, n_iter, body, 0)
        for c in write_copies(n_iter - 1, (n_iter - 1) % 2):
            c.wait()

    return k


def kernel(voc, char_table, word_table):
    b, _, l = voc.shape
    n_tokens = b * l
    idx_char = voc[:, 0, :].reshape(n_tokens // IPR, IPR).astype(jnp.int32)
    idx_word = voc[:, 1, :].reshape(n_tokens // IPR, IPR).astype(jnp.int32)
    out = _make_sc_kernel(n_tokens)(idx_char, idx_word, char_table, word_table)
    return out.reshape(b, l, OUT_DIM)


# speedup vs baseline: 7.7374x; 1.0506x over previous
import functools

import jax
import jax.numpy as jnp
from jax import lax
from jax.experimental import pallas as pl
from jax.experimental.pallas import tpu as pltpu
from jax.experimental.pallas import tpu_sc as plsc

CHAR_DIM = 32
WORD_DIM = 32
OUT_DIM = CHAR_DIM + WORD_DIM
IPR = 128
CHUNK_ROWS = 4
CHUNK = CHUNK_ROWS * IPR
NSLOTS = 3


@functools.lru_cache(maxsize=None)
def _make_sc_kernel(n_tokens: int):
    info = plsc.get_sparse_core_info()
    nw = info.num_cores * info.num_subcores
    assert n_tokens % (nw * CHUNK) == 0
    rows_per_w = (n_tokens // IPR) // nw
    n_iter = rows_per_w // CHUNK_ROWS
    assert n_iter >= NSLOTS
    nc = info.num_cores

    mesh = plsc.VectorSubcoreMesh(core_axis_name="c", subcore_axis_name="s")

    @functools.partial(
        pl.kernel,
        mesh=mesh,
        out_type=jax.ShapeDtypeStruct((n_tokens, OUT_DIM), jnp.float32),
        compiler_params=pltpu.CompilerParams(use_tc_tiling_on_sc=False),
        scratch_types=[
            pltpu.VMEM((NSLOTS, CHUNK_ROWS, IPR), jnp.int32),
            pltpu.VMEM((NSLOTS, CHUNK_ROWS, IPR), jnp.int32),
            pltpu.VMEM((NSLOTS, CHUNK, CHAR_DIM), jnp.float32),
            pltpu.VMEM((NSLOTS, CHUNK, WORD_DIM), jnp.float32),
            pltpu.SemaphoreType.DMA((NSLOTS,)),
            pltpu.SemaphoreType.DMA((NSLOTS,)),
        ],
    )
    def k(idx_char_hbm, idx_word_hbm, char_hbm, word_hbm, out_hbm,
          ic_v, iw_v, cb_v, wb_v, sem_g, sem_w):
        wid = lax.axis_index("s") * nc + lax.axis_index("c")
        row_base = wid * rows_per_w
        tok_base = row_base * IPR

        def gather_copies(slot):
            copies = []
            for g in range(CHUNK_ROWS):
                rows = pl.ds(g * IPR, IPR)
                copies.append(pltpu.make_async_copy(
                    char_hbm.at[ic_v.at[slot, g]],
                    cb_v.at[slot, rows], sem_g.at[slot]))
                copies.append(pltpu.make_async_copy(
                    word_hbm.at[iw_v.at[slot, g]],
                    wb_v.at[slot, rows], sem_g.at[slot]))
            return copies

        def issue_gathers(chunk_i, slot):
            row0 = row_base + chunk_i * CHUNK_ROWS
            pltpu.sync_copy(idx_char_hbm.at[pl.ds(row0, CHUNK_ROWS)],
                            ic_v.at[slot])
            pltpu.sync_copy(idx_word_hbm.at[pl.ds(row0, CHUNK_ROWS)],
                            iw_v.at[slot])
            for c in gather_copies(slot):
                c.start()

        def write_copies(chunk_i, slot):
            tok0 = tok_base + chunk_i * CHUNK
            return [
                pltpu.make_async_copy(
                    cb_v.at[slot],
                    out_hbm.at[pl.ds(tok0, CHUNK), pl.ds(0, CHAR_DIM)],
                    sem_w.at[slot]),
                pltpu.make_async_copy(
                    wb_v.at[slot],
                    out_hbm.at[pl.ds(tok0, CHUNK), pl.ds(CHAR_DIM, WORD_DIM)],
                    sem_w.at[slot]),
            ]

        for p in range(NSLOTS - 1):
            issue_gathers(p, p)

        def body(i, carry):
            slot = lax.rem(i, NSLOTS)
            for c in gather_copies(slot):
                c.wait()
            for c in write_copies(i, slot):
                c.start()

            @pl.when(i + NSLOTS - 1 < n_iter)
            def _issue_ahead():
                nslot = lax.rem(i + NSLOTS - 1, NSLOTS)

                @pl.when(i > 0)
                def _drain_stale_write():
                    for c in write_copies(i - 1, nslot):
                        c.wait()

                issue_gathers(i + NSLOTS - 1, nslot)

            return carry

        lax.fori_loop(0, n_iter, body, 0)
        for tail in range(NSLOTS, 0, -1):
            for c in write_copies(n_iter - tail, (n_iter - tail) % NSLOTS):
                c.wait()

    return k


def kernel(voc, char_table, word_table):
    b, _, l = voc.shape
    n_tokens = b * l
    idx_char = voc[:, 0, :].reshape(n_tokens // IPR, IPR).astype(jnp.int32)
    idx_word = voc[:, 1, :].reshape(n_tokens // IPR, IPR).astype(jnp.int32)
    out = _make_sc_kernel(n_tokens)(idx_char, idx_word, char_table, word_table)
    return out.reshape(b, l, OUT_DIM)
